# baseline (speedup 1.0000x reference)
import functools

import jax
import jax.numpy as jnp
from jax import lax
from jax.experimental import pallas as pl
from jax.experimental.pallas import tpu as pltpu
from jax.experimental.pallas import tpu_sc as plsc

VOCAB = 100000
VOCAB_P = 100352
EMBED = 300
NCLS = 6
BATCH = 4096
SEQ = 200
SPAN = VOCAB_P // 32

DPAD = 8
OSTRIDE = DPAD + 1
NCORES = 2
NSUB = 16
NW = NCORES * NSUB
COLS_PER_W = BATCH // NW
PAIRS = COLS_PER_W // 2
LC = 25
NG = SEQ // LC

VBLK = 8192


def _matmul_body(tt_ref, w_ref, o_ref):
    o_ref[...] = lax.dot_general(
        w_ref[...], tt_ref[...], (((0,), (0,)), ((), ())),
        preferred_element_type=jnp.float32)


def _table_times_w(table_t, wt):
    return pl.pallas_call(
        _matmul_body,
        grid=((VOCAB_P + VBLK - 1) // VBLK,),
        in_specs=[
            pl.BlockSpec((EMBED, VBLK), lambda i: (0, i)),
            pl.BlockSpec((EMBED, DPAD), lambda i: (0, 0)),
        ],
        out_specs=pl.BlockSpec((DPAD, VBLK), lambda i: (0, i)),
        out_shape=jax.ShapeDtypeStruct((DPAD, VOCAB_P), jnp.float32),
    )(table_t, wt)


@functools.partial(
    pl.kernel,
    out_type=jax.ShapeDtypeStruct((VOCAB_P * DPAD,), jnp.float32),
    mesh=plsc.VectorSubcoreMesh(
        core_axis_name="c", subcore_axis_name="s",
        num_cores=NCORES, num_subcores=NSUB),
    scratch_types=[
        pltpu.VMEM((DPAD, SPAN), jnp.float32),
        pltpu.VMEM((SPAN * OSTRIDE,), jnp.float32),
        pltpu.VMEM((SPAN * DPAD,), jnp.float32),
    ],
    compiler_params=pltpu.CompilerParams(use_tc_tiling_on_sc=False,
                                         needs_layout_passes=False),
)
def _sc_transpose(twt_hbm, out_hbm, buf, outb, outc):
    wid = lax.axis_index("s") * NCORES + lax.axis_index("c")
    base = wid * SPAN
    pltpu.sync_copy(twt_hbm.at[:, pl.ds(base, SPAN)], buf)
    lanes = lax.iota(jnp.int32, 16)
    v9 = lanes * OSTRIDE

    def body(i, _):
        b9 = v9 + i * (16 * OSTRIDE)
        for r in range(DPAD):
            plsc.store_scatter(outb, [b9 + r], buf[r, pl.ds(i * 16, 16)])
        return 0

    lax.fori_loop(0, SPAN // 16, body, 0)

    def compact(k, _):
        t = lanes + k * 16
        outc[pl.ds(k * 16, 16)] = plsc.load_gather(outb, [t + t // DPAD])
        return 0

    lax.fori_loop(0, SPAN * DPAD // 16, compact, 0)
    pltpu.sync_copy(outc, out_hbm.at[pl.ds(base * DPAD, SPAN * DPAD)])


@functools.partial(
    pl.kernel,
    out_type=jax.ShapeDtypeStruct((BATCH * DPAD,), jnp.float32),
    mesh=plsc.VectorSubcoreMesh(
        core_axis_name="c", subcore_axis_name="s",
        num_cores=NCORES, num_subcores=NSUB),
    scratch_types=[
        pltpu.VMEM((SEQ, COLS_PER_W), jnp.int32),
        pltpu.VMEM((2, LC, COLS_PER_W, DPAD), jnp.float32),
        pltpu.VMEM((COLS_PER_W * DPAD,), jnp.float32),
        pltpu.VMEM((16,), jnp.float32),
        pltpu.SemaphoreType.DMA,
        pltpu.SemaphoreType.DMA,
    ],
    compiler_params=pltpu.CompilerParams(use_tc_tiling_on_sc=False,
                                         needs_layout_passes=False),
)
def _sc_pool(tw_hbm, xt_hbm, bias_hbm, out_hbm,
             idx_v, gbuf, out_v, bias_v, sem0, sem1):
    wid = lax.axis_index("s") * NCORES + lax.axis_index("c")
    base = wid * COLS_PER_W

    pltpu.sync_copy(xt_hbm.at[:, pl.ds(base, COLS_PER_W)], idx_v)
    pltpu.sync_copy(bias_hbm, bias_v)
    bias = bias_v[...]
    sems = (sem0, sem1)
    lanes = lax.iota(jnp.int32, 16)
    row2 = lanes // 8
    col8 = lanes % 8
    zero16 = jnp.zeros((16,), jnp.float32)
    for t in range(PAIRS):
        out_v[pl.ds(t * 16, 16)] = zero16

    def fire(g, slot):
        for c in range(LC):
            pltpu.async_copy(tw_hbm.at[idx_v.at[g * LC + c]],
                             gbuf.at[slot, c], sems[slot])

    def drain_accum(g, slot):
        for c in range(LC):
            pltpu.make_async_copy(tw_hbm.at[idx_v.at[g * LC + c]],
                                  gbuf.at[slot, c], sems[slot]).wait()
        for bg in range(PAIRS // 8):
            rows = [row2 + (bg * 16 + 2 * p) for p in range(8)]

            def step(c, accs, rows=rows, slot=slot):
                chunk = gbuf.at[slot, c]
                return tuple(
                    accs[p] + plsc.load_gather(chunk, [rows[p], col8])
                    for p in range(8))

            accs = lax.fori_loop(0, LC, step, tuple(zero16
                                                    for _ in range(8)))
            for p in range(8):
                plsc.addupdate(out_v.at[pl.ds((bg * 8 + p) * 16, 16)],
                               accs[p])

    fire(0, 0)

    def body(i, _):
        g = i * 2
        fire(g + 1, 1)
        drain_accum(g, 0)

        @pl.when(g + 2 < NG)
        def _():
            fire(g + 2, 0)

        drain_accum(g + 1, 1)
        return 0

    lax.fori_loop(0, NG // 2, body, 0)
    for t in range(PAIRS):
        v = out_v[pl.ds(t * 16, 16)]
        out_v[pl.ds(t * 16, 16)] = v * (1.0 / SEQ) + bias
    pltpu.sync_copy(out_v,
                    out_hbm.at[pl.ds(base * DPAD, COLS_PER_W * DPAD)])


def kernel(x, table, W, b):
    xt = x.astype(jnp.int32).T
    wt = jnp.zeros((EMBED, DPAD), jnp.float32).at[:, :NCLS].set(W.T)
    bias = (jnp.zeros((16,), jnp.float32)
            .at[:NCLS].set(b).at[8:8 + NCLS].set(b))
    tw_t = _table_times_w(table.T, wt)
    tw = _sc_transpose(tw_t).reshape(VOCAB_P, DPAD)
    out = _sc_pool(tw, xt, bias)
    return out.reshape(BATCH, DPAD)[:, :NCLS]
